# Initial kernel scaffold; baseline (speedup 1.0000x reference)
import jax
import jax.numpy as jnp
from jax.experimental import pallas as pl


def kernel(x):
    raise NotImplementedError("write your pallas kernel here")


# speedup vs baseline: 1.4208x; 1.4208x over previous
import jax
import jax.numpy as jnp
from jax import lax
from jax.experimental import pallas as pl
from jax.experimental.pallas import tpu as pltpu
from jax.experimental.pallas import tpu_sc as plsc

L = 16
NW = 32
ROWS = 64
N = 32768
V = N // L
LOGV = 11


def _vsort(v):
    return jnp.sort(v)


def _sort_body(x_hbm, out_hbm, buf):
    cid = lax.axis_index("c")
    sid = lax.axis_index("s")
    wid = sid * 2 + cid

    def vld(i):
        return buf[pl.ds(i * L, L)]

    def vst(i, v):
        buf[pl.ds(i * L, L)] = v

    def do_row(r, carry):
        row = wid + r * NW
        pltpu.sync_copy(x_hbm.at[row], buf)

        def lvl0(j, c):
            a = _vsort(vld(2 * j))
            b = _vsort(vld(2 * j + 1))
            rb = lax.rev(b, (0,))
            lo = jnp.minimum(a, rb)
            hi = jnp.maximum(a, rb)
            vst(2 * j, _vsort(lo))
            vst(2 * j + 1, _vsort(hi))
            return c

        lax.fori_loop(0, V // 2, lvl0, 0)

        for k in range(1, LOGV):
            R = 1 << k

            def stage1(j, c, k=k, R=R):
                m = j >> k
                i = j & (R - 1)
                ia = (m << (k + 1)) + i
                ib = (m << (k + 1)) + 2 * R - 1 - i
                a = vld(ia)
                rb = lax.rev(vld(ib), (0,))
                vst(ia, jnp.minimum(a, rb))
                vst(ib, lax.rev(jnp.maximum(a, rb), (0,)))
                return c

            lax.fori_loop(0, V // 2, stage1, 0)

            d = R // 2
            while d >= 2:
                ld = d.bit_length() - 1

                def stage(j, c, ld=ld, d=d):
                    p = ((j >> ld) << (ld + 1)) + (j & (d - 1))
                    q = p + d
                    a = vld(p)
                    b = vld(q)
                    vst(p, jnp.minimum(a, b))
                    vst(q, jnp.maximum(a, b))
                    return c

                lax.fori_loop(0, V // 2, stage, 0)
                d //= 2

            def last(j, c):
                a = vld(2 * j)
                b = vld(2 * j + 1)
                lo = jnp.minimum(a, b)
                hi = jnp.maximum(a, b)
                vst(2 * j, _vsort(lo))
                vst(2 * j + 1, _vsort(hi))
                return c

            lax.fori_loop(0, V // 2, last, 0)

        pltpu.sync_copy(buf, out_hbm.at[row])
        return carry

    lax.fori_loop(0, ROWS // NW, do_row, 0)


@jax.jit
def kernel(x):
    mesh = plsc.VectorSubcoreMesh(core_axis_name="c", subcore_axis_name="s")
    out = pl.kernel(
        _sort_body,
        out_type=jax.ShapeDtypeStruct((ROWS, N), jnp.float32),
        mesh=mesh,
        scratch_types=[pltpu.VMEM((N,), jnp.float32)],
        compiler_params=pltpu.CompilerParams(needs_layout_passes=False),
    )(x)
    return out


# speedup vs baseline: 5.5447x; 3.9026x over previous
import jax
import jax.numpy as jnp
from jax import lax
from jax.experimental import pallas as pl
from jax.experimental.pallas import tpu as pltpu
from jax.experimental.pallas import tpu_sc as plsc

L = 16
NW = 32
ROWS = 64
N = 32768
V = N // L
LOGV = 11
P0_LEVELS = 4
FG = 1 << P0_LEVELS


def _vsort(v):
    return jnp.sort(v)


def _vrev(v):
    return lax.rev(v, (0,))


def _reg_stages(vals, dists):
    n = len(vals)
    for d in dists:
        for s in range(0, n, 2 * d):
            for i in range(d):
                a = vals[s + i]
                b = vals[s + i + d]
                vals[s + i] = jnp.minimum(a, b)
                vals[s + i + d] = jnp.maximum(a, b)


def _reg_merge(vals):
    r = len(vals) // 2
    c = vals[:r] + [_vrev(v) for v in vals[r:][::-1]]
    dists = []
    d = r
    while d >= 1:
        dists.append(d)
        d //= 2
    _reg_stages(c, dists)
    return [_vsort(v) for v in c]


def _sort_body(x_hbm, out_hbm, buf):
    cid = lax.axis_index("c")
    sid = lax.axis_index("s")
    wid = sid * 2 + cid

    def vld(i):
        return buf[pl.ds(i * L, L)]

    def vst(i, v):
        buf[pl.ds(i * L, L)] = v

    def do_row(r, carry):
        row = wid + r * NW
        pltpu.sync_copy(x_hbm.at[row], buf)

        def p0(m, c):
            base = m * FG
            vals = [_vsort(vld(base + j)) for j in range(FG)]
            for k in range(P0_LEVELS):
                sz = 1 << (k + 1)
                out = []
                for g in range(FG // sz):
                    out.extend(_reg_merge(vals[g * sz:(g + 1) * sz]))
                vals = out
            for j in range(FG):
                vst(base + j, vals[j])
            return c

        lax.fori_loop(0, V // FG, p0, 0)

        for k in range(P0_LEVELS, LOGV):
            R = 1 << k

            def stage1(j, c, k=k, R=R):
                m = j >> k
                i = j & (R - 1)
                ia = (m << (k + 1)) + i
                ib = (m << (k + 1)) + 2 * R - 1 - i
                a = vld(ia)
                rb = _vrev(vld(ib))
                vst(ia, jnp.minimum(a, rb))
                vst(ib, _vrev(jnp.maximum(a, rb)))
                return c

            lax.fori_loop(0, V // 2, stage1, 0, unroll=2)

            dists = []
            d = R // 2
            while d >= FG:
                dists.append(d)
                d //= 2
            while dists:
                take = 3 if len(dists) >= 3 else len(dists)
                chunk, dists = dists[:take], dists[take:]
                stride = chunk[-1]
                ls = stride.bit_length() - 1
                G = 1 << take
                block = 2 * chunk[0]
                lb = block.bit_length() - 1

                def fused(it, c, stride=stride, ls=ls, G=G, lb=lb):
                    base = ((it >> ls) << lb) + (it & (stride - 1))
                    g = [vld(base + j * stride) for j in range(G)]
                    _reg_stages(g, [1 << t for t in range(take - 1, -1, -1)])
                    for j in range(G):
                        vst(base + j * stride, g[j])
                    return c

                lax.fori_loop(0, V // G, fused, 0)

            def last(m, c):
                base = m * FG
                g = [vld(base + j) for j in range(FG)]
                _reg_stages(g, [8, 4, 2, 1])
                for j in range(FG):
                    vst(base + j, _vsort(g[j]))
                return c

            lax.fori_loop(0, V // FG, last, 0)

        pltpu.sync_copy(buf, out_hbm.at[row])
        return carry

    lax.fori_loop(0, ROWS // NW, do_row, 0)


@jax.jit
def kernel(x):
    mesh = plsc.VectorSubcoreMesh(core_axis_name="c", subcore_axis_name="s")
    out = pl.kernel(
        _sort_body,
        out_type=jax.ShapeDtypeStruct((ROWS, N), jnp.float32),
        mesh=mesh,
        scratch_types=[pltpu.VMEM((N,), jnp.float32)],
        compiler_params=pltpu.CompilerParams(needs_layout_passes=False),
    )(x)
    return out


# speedup vs baseline: 7.9401x; 1.4320x over previous
import jax
import jax.numpy as jnp
from jax import lax
from jax.experimental import pallas as pl
from jax.experimental.pallas import tpu as pltpu
from jax.experimental.pallas import tpu_sc as plsc

L = 16
NW = 32
ROWS = 64
N = 32768
V = N // L
LOGV = 11
P0_LEVELS = 4
FG = 1 << P0_LEVELS


def _vsort(v):
    return jnp.sort(v)


def _vrev(v):
    return lax.rev(v, (0,))


def _reg_stages(vals, dists):
    n = len(vals)
    for d in dists:
        for s in range(0, n, 2 * d):
            for i in range(d):
                a = vals[s + i]
                b = vals[s + i + d]
                vals[s + i] = jnp.minimum(a, b)
                vals[s + i + d] = jnp.maximum(a, b)


def _reg_merge(vals):
    r = len(vals) // 2
    c = vals[:r] + [_vrev(v) for v in vals[r:][::-1]]
    dists = []
    d = r
    while d >= 1:
        dists.append(d)
        d //= 2
    _reg_stages(c, dists)
    return [_vsort(v) for v in c]


def _sort_body(x_hbm, out_hbm, buf):
    cid = lax.axis_index("c")
    sid = lax.axis_index("s")
    wid = sid * 2 + cid

    def vld(i):
        return buf[pl.ds(i * L, L)]

    def vst(i, v):
        buf[pl.ds(i * L, L)] = v

    def do_row(r, carry):
        row = wid + r * NW
        pltpu.sync_copy(x_hbm.at[row], buf)

        def p0(m, c):
            base = m * FG
            vals = [_vsort(vld(base + j)) for j in range(FG)]
            for k in range(P0_LEVELS):
                sz = 1 << (k + 1)
                out = []
                for g in range(FG // sz):
                    out.extend(_reg_merge(vals[g * sz:(g + 1) * sz]))
                vals = out
            for j in range(FG):
                vst(base + j, vals[j])
            return c

        lax.fori_loop(0, V // FG, p0, 0)

        for k in range(P0_LEVELS, LOGV):
            R = 1 << k

            m = min(8, 1 << (k - P0_LEVELS))
            s = R // m
            ls = s.bit_length() - 1
            rs_dists = [1 << t for t in range((m.bit_length() - 1) - 1, -1, -1)]

            def refl(it, c, k=k, R=R, m=m, s=s, ls=ls, rs_dists=rs_dists):
                blk = it >> ls
                o = it & (s - 1)
                base = (blk << (k + 1)) + o
                top = (blk << (k + 1)) + 2 * R - 1 - o
                lo = [vld(base + j * s) for j in range(m)]
                hi = []
                for j in range(m):
                    rb = _vrev(vld(top - j * s))
                    a = lo[j]
                    lo[j] = jnp.minimum(a, rb)
                    hi.append(jnp.maximum(a, rb))
                hlist = [_vrev(hi[m - 1 - jp]) for jp in range(m)]
                _reg_stages(lo, rs_dists)
                _reg_stages(hlist, rs_dists)
                for j in range(m):
                    vst(base + j * s, lo[j])
                for jp in range(m):
                    vst(top - (m - 1 - jp) * s, hlist[jp])
                return c

            lax.fori_loop(0, V // (2 * m), refl, 0, unroll=2 if m == 1 else 1)

            dists = []
            d = R // (2 * m)
            while d >= FG:
                dists.append(d)
                d //= 2
            while dists:
                take = 3 if len(dists) >= 3 else len(dists)
                chunk, dists = dists[:take], dists[take:]
                stride = chunk[-1]
                ls = stride.bit_length() - 1
                G = 1 << take
                block = 2 * chunk[0]
                lb = block.bit_length() - 1

                def fused(it, c, stride=stride, ls=ls, G=G, lb=lb):
                    base = ((it >> ls) << lb) + (it & (stride - 1))
                    g = [vld(base + j * stride) for j in range(G)]
                    _reg_stages(g, [1 << t for t in range(take - 1, -1, -1)])
                    for j in range(G):
                        vst(base + j * stride, g[j])
                    return c

                lax.fori_loop(0, V // G, fused, 0)

            def last(m, c):
                base = m * FG
                g = [vld(base + j) for j in range(FG)]
                _reg_stages(g, [8, 4, 2, 1])
                for j in range(FG):
                    vst(base + j, _vsort(g[j]))
                return c

            lax.fori_loop(0, V // FG, last, 0)

        pltpu.sync_copy(buf, out_hbm.at[row])
        return carry

    lax.fori_loop(0, ROWS // NW, do_row, 0)


@jax.jit
def kernel(x):
    mesh = plsc.VectorSubcoreMesh(core_axis_name="c", subcore_axis_name="s")
    out = pl.kernel(
        _sort_body,
        out_type=jax.ShapeDtypeStruct((ROWS, N), jnp.float32),
        mesh=mesh,
        scratch_types=[pltpu.VMEM((N,), jnp.float32)],
        compiler_params=pltpu.CompilerParams(needs_layout_passes=False),
    )(x)
    return out


# speedup vs baseline: 8.4185x; 1.0603x over previous
import jax
import jax.numpy as jnp
from jax import lax
from jax.experimental import pallas as pl
from jax.experimental.pallas import tpu as pltpu
from jax.experimental.pallas import tpu_sc as plsc

L = 16
NW = 32
ROWS = 64
N = 32768
V = N // L
LOGV = 11
P0_LEVELS = 4
FG = 1 << P0_LEVELS


def _vsort(v):
    return jnp.sort(v)


def _vrev(v):
    return lax.rev(v, (0,))


def _reg_stages(vals, dists):
    n = len(vals)
    for d in dists:
        for s in range(0, n, 2 * d):
            for i in range(d):
                a = vals[s + i]
                b = vals[s + i + d]
                vals[s + i] = jnp.minimum(a, b)
                vals[s + i + d] = jnp.maximum(a, b)


def _reg_merge(vals):
    r = len(vals) // 2
    c = vals[:r] + [_vrev(v) for v in vals[r:][::-1]]
    dists = []
    d = r
    while d >= 1:
        dists.append(d)
        d //= 2
    _reg_stages(c, dists)
    return [_vsort(v) for v in c]


def _row_sort(buf):
    def vld(i):
        return buf[pl.ds(i * L, L)]

    def vst(i, v):
        buf[pl.ds(i * L, L)] = v

    def p0(m, c):
        base = m * FG
        vals = [_vsort(vld(base + j)) for j in range(FG)]
        for k in range(P0_LEVELS):
            sz = 1 << (k + 1)
            out = []
            for g in range(FG // sz):
                out.extend(_reg_merge(vals[g * sz:(g + 1) * sz]))
            vals = out
        for j in range(FG):
            vst(base + j, vals[j])
        return c

    lax.fori_loop(0, V // FG, p0, 0)

    for k in range(P0_LEVELS, LOGV):
        R = 1 << k

        m = min(8, 1 << (k - P0_LEVELS))
        s = R // m
        ls = s.bit_length() - 1
        rs_dists = [1 << t for t in range((m.bit_length() - 1) - 1, -1, -1)]

        def refl(it, c, k=k, R=R, m=m, s=s, ls=ls, rs_dists=rs_dists):
            blk = it >> ls
            o = it & (s - 1)
            base = (blk << (k + 1)) + o
            top = (blk << (k + 1)) + 2 * R - 1 - o
            lo = [vld(base + j * s) for j in range(m)]
            hi = []
            for j in range(m):
                rb = _vrev(vld(top - j * s))
                a = lo[j]
                lo[j] = jnp.minimum(a, rb)
                hi.append(jnp.maximum(a, rb))
            hlist = [_vrev(hi[m - 1 - jp]) for jp in range(m)]
            _reg_stages(lo, rs_dists)
            _reg_stages(hlist, rs_dists)
            for j in range(m):
                vst(base + j * s, lo[j])
            for jp in range(m):
                vst(top - (m - 1 - jp) * s, hlist[jp])
            return c

        lax.fori_loop(0, V // (2 * m), refl, 0, unroll=2 if m == 1 else 1)

        dists = []
        d = R // (2 * m)
        while d >= FG:
            dists.append(d)
            d //= 2

        gf, fdists = FG, [8, 4, 2, 1]
        if dists == [FG]:
            gf, fdists, dists = 2 * FG, [16, 8, 4, 2, 1], []

        while dists:
            take = 3 if len(dists) >= 3 else len(dists)
            chunk, dists = dists[:take], dists[take:]
            stride = chunk[-1]
            lss = stride.bit_length() - 1
            G = 1 << take
            block = 2 * chunk[0]
            lb = block.bit_length() - 1

            def fused(it, c, stride=stride, lss=lss, G=G, lb=lb, take=take):
                base = ((it >> lss) << lb) + (it & (stride - 1))
                g = [vld(base + j * stride) for j in range(G)]
                _reg_stages(g, [1 << t for t in range(take - 1, -1, -1)])
                for j in range(G):
                    vst(base + j * stride, g[j])
                return c

            lax.fori_loop(0, V // G, fused, 0)

        def last(mm, c, gf=gf, fdists=fdists):
            base = mm * gf
            g = [vld(base + j) for j in range(gf)]
            _reg_stages(g, fdists)
            for j in range(gf):
                vst(base + j, _vsort(g[j]))
            return c

        lax.fori_loop(0, V // gf, last, 0)


def _sort_body(x_hbm, out_hbm, buf_a, buf_b, in_a, in_b, out_a, out_b):
    cid = lax.axis_index("c")
    sid = lax.axis_index("s")
    wid = sid * 2 + cid
    row0 = wid
    row1 = wid + NW

    cp_in0 = pltpu.make_async_copy(x_hbm.at[row0], buf_a, in_a)
    cp_in1 = pltpu.make_async_copy(x_hbm.at[row1], buf_b, in_b)
    cp_in0.start()
    cp_in1.start()

    cp_in0.wait()
    _row_sort(buf_a)
    cp_out0 = pltpu.make_async_copy(buf_a, out_hbm.at[row0], out_a)
    cp_out0.start()

    cp_in1.wait()
    _row_sort(buf_b)
    cp_out1 = pltpu.make_async_copy(buf_b, out_hbm.at[row1], out_b)
    cp_out1.start()

    cp_out0.wait()
    cp_out1.wait()


@jax.jit
def kernel(x):
    mesh = plsc.VectorSubcoreMesh(core_axis_name="c", subcore_axis_name="s")
    out = pl.kernel(
        _sort_body,
        out_type=jax.ShapeDtypeStruct((ROWS, N), jnp.float32),
        mesh=mesh,
        scratch_types=[
            pltpu.VMEM((N,), jnp.float32),
            pltpu.VMEM((N,), jnp.float32),
            pltpu.SemaphoreType.DMA,
            pltpu.SemaphoreType.DMA,
            pltpu.SemaphoreType.DMA,
            pltpu.SemaphoreType.DMA,
        ],
        compiler_params=pltpu.CompilerParams(needs_layout_passes=False),
    )(x)
    return out


# speedup vs baseline: 8.9244x; 1.0601x over previous
import jax
import jax.numpy as jnp
from jax import lax
from jax.experimental import pallas as pl
from jax.experimental.pallas import tpu as pltpu
from jax.experimental.pallas import tpu_sc as plsc

L = 16
NW = 32
ROWS = 64
N = 32768
V = N // L
LOGV = 11
P0_LEVELS = 4
FG = 1 << P0_LEVELS


def _vsort(v):
    return jnp.sort(v)


def _vrev(v):
    return lax.rev(v, (0,))


def _reg_stages(vals, dists):
    n = len(vals)
    for d in dists:
        for s in range(0, n, 2 * d):
            for i in range(d):
                a = vals[s + i]
                b = vals[s + i + d]
                vals[s + i] = jnp.minimum(a, b)
                vals[s + i + d] = jnp.maximum(a, b)


def _reg_merge(vals):
    r = len(vals) // 2
    c = vals[:r] + [_vrev(v) for v in vals[r:][::-1]]
    dists = []
    d = r
    while d >= 1:
        dists.append(d)
        d //= 2
    _reg_stages(c, dists)
    return [_vsort(v) for v in c]


def _row_sort(buf):
    def vld(i):
        return buf[pl.ds(i * L, L)]

    def vst(i, v):
        buf[pl.ds(i * L, L)] = v

    def p0(m, c):
        base = m * FG
        vals = [_vsort(vld(base + j)) for j in range(FG)]
        for k in range(P0_LEVELS):
            sz = 1 << (k + 1)
            out = []
            for g in range(FG // sz):
                out.extend(_reg_merge(vals[g * sz:(g + 1) * sz]))
            vals = out
        for j in range(FG):
            vst(base + j, vals[j])
        return c

    lax.fori_loop(0, V // FG, p0, 0)

    for k in range(P0_LEVELS, LOGV):
        R = 1 << k

        m = min(8, 1 << (k - P0_LEVELS))
        s = R // m
        ls = s.bit_length() - 1
        rs_dists = [1 << t for t in range((m.bit_length() - 1) - 1, -1, -1)]

        def refl(it, c, k=k, R=R, m=m, s=s, ls=ls, rs_dists=rs_dists):
            blk = it >> ls
            o = it & (s - 1)
            base = (blk << (k + 1)) + o
            top = (blk << (k + 1)) + 2 * R - 1 - o
            lo = [vld(base + j * s) for j in range(m)]
            hi = []
            for j in range(m):
                rb = _vrev(vld(top - j * s))
                a = lo[j]
                lo[j] = jnp.minimum(a, rb)
                hi.append(jnp.maximum(a, rb))
            hlist = [_vrev(hi[m - 1 - jp]) for jp in range(m)]
            _reg_stages(lo, rs_dists)
            _reg_stages(hlist, rs_dists)
            for j in range(m):
                vst(base + j * s, lo[j])
            for jp in range(m):
                vst(top - (m - 1 - jp) * s, hlist[jp])
            return c

        lax.fori_loop(0, V // (2 * m), refl, 0, unroll=2 if m == 1 else 1)

        dists = []
        d = R // (2 * m)
        while d >= FG:
            dists.append(d)
            d //= 2

        gf, fdists = FG, [8, 4, 2, 1]
        if dists == [FG]:
            gf, fdists, dists = 2 * FG, [16, 8, 4, 2, 1], []

        while dists:
            take = 3 if len(dists) >= 3 else len(dists)
            chunk, dists = dists[:take], dists[take:]
            stride = chunk[-1]
            lss = stride.bit_length() - 1
            G = 1 << take
            block = 2 * chunk[0]
            lb = block.bit_length() - 1

            def fused(it, c, stride=stride, lss=lss, G=G, lb=lb, take=take):
                base = ((it >> lss) << lb) + (it & (stride - 1))
                g = [vld(base + j * stride) for j in range(G)]
                _reg_stages(g, [1 << t for t in range(take - 1, -1, -1)])
                for j in range(G):
                    vst(base + j * stride, g[j])
                return c

            lax.fori_loop(0, V // G, fused, 0)

        def last(mm, c, gf=gf, fdists=fdists):
            base = mm * gf
            g = [vld(base + j) for j in range(gf)]
            _reg_stages(g, fdists)
            for j in range(gf):
                vst(base + j, _vsort(g[j]))
            return c

        lax.fori_loop(0, V // gf, last, 0, unroll=2 if gf == FG else 1)


def _sort_body(x_hbm, out_hbm, buf_a, buf_b, in_a, in_b, out_a, out_b):
    cid = lax.axis_index("c")
    sid = lax.axis_index("s")
    wid = sid * 2 + cid
    row0 = wid
    row1 = wid + NW

    cp_in0 = pltpu.make_async_copy(x_hbm.at[row0], buf_a, in_a)
    cp_in1 = pltpu.make_async_copy(x_hbm.at[row1], buf_b, in_b)
    cp_in0.start()
    cp_in1.start()

    cp_in0.wait()
    _row_sort(buf_a)
    cp_out0 = pltpu.make_async_copy(buf_a, out_hbm.at[row0], out_a)
    cp_out0.start()

    cp_in1.wait()
    _row_sort(buf_b)
    cp_out1 = pltpu.make_async_copy(buf_b, out_hbm.at[row1], out_b)
    cp_out1.start()

    cp_out0.wait()
    cp_out1.wait()


@jax.jit
def kernel(x):
    mesh = plsc.VectorSubcoreMesh(core_axis_name="c", subcore_axis_name="s")
    out = pl.kernel(
        _sort_body,
        out_type=jax.ShapeDtypeStruct((ROWS, N), jnp.float32),
        mesh=mesh,
        scratch_types=[
            pltpu.VMEM((N,), jnp.float32),
            pltpu.VMEM((N,), jnp.float32),
            pltpu.SemaphoreType.DMA,
            pltpu.SemaphoreType.DMA,
            pltpu.SemaphoreType.DMA,
            pltpu.SemaphoreType.DMA,
        ],
        compiler_params=pltpu.CompilerParams(needs_layout_passes=False),
    )(x)
    return out


# speedup vs baseline: 10.1696x; 1.1395x over previous
import jax
import jax.numpy as jnp
from jax import lax
from jax.experimental import pallas as pl
from jax.experimental.pallas import tpu as pltpu
from jax.experimental.pallas import tpu_sc as plsc

L = 16
NW = 32
ROWS = 64
N = 32768
V = N // L
LOGV = 11
P0_LEVELS = 5
P0G = 1 << P0_LEVELS
FG = 16


def _vsort(v):
    return jnp.sort(v)


def _vrev(v):
    return lax.rev(v, (0,))


def _reg_stages(vals, dists):
    n = len(vals)
    for d in dists:
        for s in range(0, n, 2 * d):
            for i in range(d):
                a = vals[s + i]
                b = vals[s + i + d]
                vals[s + i] = jnp.minimum(a, b)
                vals[s + i + d] = jnp.maximum(a, b)


def _reg_merge(vals):
    r = len(vals) // 2
    c = vals[:r] + [_vrev(v) for v in vals[r:][::-1]]
    dists = []
    d = r
    while d >= 1:
        dists.append(d)
        d //= 2
    _reg_stages(c, dists)
    return [_vsort(v) for v in c]


def _row_sort(buf):
    def vld(i):
        return buf[pl.ds(i * L, L)]

    def vst(i, v):
        buf[pl.ds(i * L, L)] = v

    def p0(m, c):
        base = m * P0G
        vals = [_vsort(vld(base + j)) for j in range(P0G)]
        for k in range(P0_LEVELS):
            sz = 1 << (k + 1)
            out = []
            for g in range(P0G // sz):
                out.extend(_reg_merge(vals[g * sz:(g + 1) * sz]))
            vals = out
        for j in range(P0G):
            vst(base + j, vals[j])
        return c

    lax.fori_loop(0, V // P0G, p0, 0)

    for k in range(P0_LEVELS, LOGV):
        R = 1 << k

        m = min(8, 1 << (k - 4))
        s = R // m
        ls = s.bit_length() - 1
        rs_dists = [1 << t for t in range((m.bit_length() - 1) - 1, -1, -1)]

        def refl(it, c, k=k, R=R, m=m, s=s, ls=ls, rs_dists=rs_dists):
            blk = it >> ls
            o = it & (s - 1)
            base = (blk << (k + 1)) + o
            top = (blk << (k + 1)) + 2 * R - 1 - o
            lo = [vld(base + j * s) for j in range(m)]
            hi = []
            for j in range(m):
                rb = _vrev(vld(top - j * s))
                a = lo[j]
                lo[j] = jnp.minimum(a, rb)
                hi.append(jnp.maximum(a, rb))
            hlist = [_vrev(hi[m - 1 - jp]) for jp in range(m)]
            _reg_stages(lo, rs_dists)
            _reg_stages(hlist, rs_dists)
            for j in range(m):
                vst(base + j * s, lo[j])
            for jp in range(m):
                vst(top - (m - 1 - jp) * s, hlist[jp])
            return c

        lax.fori_loop(0, V // (2 * m), refl, 0, unroll=2 if m <= 2 else 1)

        dists = []
        d = R // (2 * m)
        while d >= FG:
            dists.append(d)
            d //= 2

        gf, fdists = FG, [8, 4, 2, 1]
        if dists == [FG]:
            gf, fdists, dists = 2 * FG, [16, 8, 4, 2, 1], []

        while dists:
            take = 3 if len(dists) >= 3 else len(dists)
            chunk, dists = dists[:take], dists[take:]
            stride = chunk[-1]
            lss = stride.bit_length() - 1
            G = 1 << take
            block = 2 * chunk[0]
            lb = block.bit_length() - 1

            def fused(it, c, stride=stride, lss=lss, G=G, lb=lb, take=take):
                base = ((it >> lss) << lb) + (it & (stride - 1))
                g = [vld(base + j * stride) for j in range(G)]
                _reg_stages(g, [1 << t for t in range(take - 1, -1, -1)])
                for j in range(G):
                    vst(base + j * stride, g[j])
                return c

            lax.fori_loop(0, V // G, fused, 0)

        def last(mm, c, gf=gf, fdists=fdists):
            base = mm * gf
            g = [vld(base + j) for j in range(gf)]
            _reg_stages(g, fdists)
            for j in range(gf):
                vst(base + j, _vsort(g[j]))
            return c

        lax.fori_loop(0, V // gf, last, 0, unroll=2 if gf == FG else 1)


def _sort_body(x_hbm, out_hbm, buf_a, buf_b, in_a, in_b, out_a, out_b):
    cid = lax.axis_index("c")
    sid = lax.axis_index("s")
    wid = sid * 2 + cid
    row0 = wid
    row1 = wid + NW

    cp_in0 = pltpu.make_async_copy(x_hbm.at[row0], buf_a, in_a)
    cp_in1 = pltpu.make_async_copy(x_hbm.at[row1], buf_b, in_b)
    cp_in0.start()
    cp_in1.start()

    cp_in0.wait()
    _row_sort(buf_a)
    cp_out0 = pltpu.make_async_copy(buf_a, out_hbm.at[row0], out_a)
    cp_out0.start()

    cp_in1.wait()
    _row_sort(buf_b)
    cp_out1 = pltpu.make_async_copy(buf_b, out_hbm.at[row1], out_b)
    cp_out1.start()

    cp_out0.wait()
    cp_out1.wait()


@jax.jit
def kernel(x):
    mesh = plsc.VectorSubcoreMesh(core_axis_name="c", subcore_axis_name="s")
    out = pl.kernel(
        _sort_body,
        out_type=jax.ShapeDtypeStruct((ROWS, N), jnp.float32),
        mesh=mesh,
        scratch_types=[
            pltpu.VMEM((N,), jnp.float32),
            pltpu.VMEM((N,), jnp.float32),
            pltpu.SemaphoreType.DMA,
            pltpu.SemaphoreType.DMA,
            pltpu.SemaphoreType.DMA,
            pltpu.SemaphoreType.DMA,
        ],
        compiler_params=pltpu.CompilerParams(needs_layout_passes=False),
    )(x)
    return out


# speedup vs baseline: 10.5607x; 1.0385x over previous
import jax
import jax.numpy as jnp
from jax import lax
from jax.experimental import pallas as pl
from jax.experimental.pallas import tpu as pltpu
from jax.experimental.pallas import tpu_sc as plsc

L = 16
NW = 32
ROWS = 64
N = 32768
V = N // L
LOGV = 11
P0_LEVELS = 5
P0G = 1 << P0_LEVELS
FG = 16


def _vsort(v):
    return jnp.sort(v)


def _vrev(v):
    return lax.rev(v, (0,))


def _reg_stages(vals, dists):
    n = len(vals)
    for d in dists:
        for s in range(0, n, 2 * d):
            for i in range(d):
                a = vals[s + i]
                b = vals[s + i + d]
                vals[s + i] = jnp.minimum(a, b)
                vals[s + i + d] = jnp.maximum(a, b)


def _reg_merge(vals):
    r = len(vals) // 2
    c = vals[:r] + [_vrev(v) for v in vals[r:][::-1]]
    dists = []
    d = r
    while d >= 1:
        dists.append(d)
        d //= 2
    _reg_stages(c, dists)
    return [_vsort(v) for v in c]


def _row_sort(buf):
    def vld(i):
        return buf[pl.ds(i * L, L)]

    def vst(i, v):
        buf[pl.ds(i * L, L)] = v

    def p0(m, c):
        base = m * P0G
        vals = [_vsort(vld(base + j)) for j in range(P0G)]
        for k in range(P0_LEVELS):
            sz = 1 << (k + 1)
            out = []
            for g in range(P0G // sz):
                out.extend(_reg_merge(vals[g * sz:(g + 1) * sz]))
            vals = out
        for j in range(P0G):
            vst(base + j, vals[j])
        return c

    lax.fori_loop(0, V // P0G, p0, 0)

    for k in range(P0_LEVELS, LOGV):
        R = 1 << k

        m = min(16, 1 << (k - 4))
        s = R // m
        ls = s.bit_length() - 1
        rs_dists = [1 << t for t in range((m.bit_length() - 1) - 1, -1, -1)]

        def refl(it, c, k=k, R=R, m=m, s=s, ls=ls, rs_dists=rs_dists):
            blk = it >> ls
            o = it & (s - 1)
            base = (blk << (k + 1)) + o
            top = (blk << (k + 1)) + 2 * R - 1 - o
            lo = [vld(base + j * s) for j in range(m)]
            hi = []
            for j in range(m):
                rb = _vrev(vld(top - j * s))
                a = lo[j]
                lo[j] = jnp.minimum(a, rb)
                hi.append(jnp.maximum(a, rb))
            hlist = [_vrev(hi[m - 1 - jp]) for jp in range(m)]
            _reg_stages(lo, rs_dists)
            _reg_stages(hlist, rs_dists)
            for j in range(m):
                vst(base + j * s, lo[j])
            for jp in range(m):
                vst(top - (m - 1 - jp) * s, hlist[jp])
            return c

        lax.fori_loop(0, V // (2 * m), refl, 0, unroll=2 if m <= 2 else 1)

        dists = []
        d = R // (2 * m)
        while d >= FG:
            dists.append(d)
            d //= 2

        gf, fdists = FG, [8, 4, 2, 1]
        if dists == [FG]:
            gf, fdists, dists = 2 * FG, [16, 8, 4, 2, 1], []

        while dists:
            take = 3 if len(dists) >= 3 else len(dists)
            chunk, dists = dists[:take], dists[take:]
            stride = chunk[-1]
            lss = stride.bit_length() - 1
            G = 1 << take
            block = 2 * chunk[0]
            lb = block.bit_length() - 1

            def fused(it, c, stride=stride, lss=lss, G=G, lb=lb, take=take):
                base = ((it >> lss) << lb) + (it & (stride - 1))
                g = [vld(base + j * stride) for j in range(G)]
                _reg_stages(g, [1 << t for t in range(take - 1, -1, -1)])
                for j in range(G):
                    vst(base + j * stride, g[j])
                return c

            lax.fori_loop(0, V // G, fused, 0)

        def last(mm, c, gf=gf, fdists=fdists):
            base = mm * gf
            g = [vld(base + j) for j in range(gf)]
            _reg_stages(g, fdists)
            for j in range(gf):
                vst(base + j, _vsort(g[j]))
            return c

        lax.fori_loop(0, V // gf, last, 0, unroll=2 if gf == FG else 1)


def _sort_body(x_hbm, out_hbm, buf_a, buf_b, in_a, in_b, out_a, out_b):
    cid = lax.axis_index("c")
    sid = lax.axis_index("s")
    wid = sid * 2 + cid
    row0 = wid
    row1 = wid + NW

    cp_in0 = pltpu.make_async_copy(x_hbm.at[row0], buf_a, in_a)
    cp_in1 = pltpu.make_async_copy(x_hbm.at[row1], buf_b, in_b)
    cp_in0.start()
    cp_in1.start()

    cp_in0.wait()
    _row_sort(buf_a)
    cp_out0 = pltpu.make_async_copy(buf_a, out_hbm.at[row0], out_a)
    cp_out0.start()

    cp_in1.wait()
    _row_sort(buf_b)
    cp_out1 = pltpu.make_async_copy(buf_b, out_hbm.at[row1], out_b)
    cp_out1.start()

    cp_out0.wait()
    cp_out1.wait()


@jax.jit
def kernel(x):
    mesh = plsc.VectorSubcoreMesh(core_axis_name="c", subcore_axis_name="s")
    out = pl.kernel(
        _sort_body,
        out_type=jax.ShapeDtypeStruct((ROWS, N), jnp.float32),
        mesh=mesh,
        scratch_types=[
            pltpu.VMEM((N,), jnp.float32),
            pltpu.VMEM((N,), jnp.float32),
            pltpu.SemaphoreType.DMA,
            pltpu.SemaphoreType.DMA,
            pltpu.SemaphoreType.DMA,
            pltpu.SemaphoreType.DMA,
        ],
        compiler_params=pltpu.CompilerParams(needs_layout_passes=False),
    )(x)
    return out


# speedup vs baseline: 10.8221x; 1.0248x over previous
import jax
import jax.numpy as jnp
from jax import lax
from jax.experimental import pallas as pl
from jax.experimental.pallas import tpu as pltpu
from jax.experimental.pallas import tpu_sc as plsc

L = 16
NW = 32
ROWS = 64
N = 32768
V = N // L
LOGV = 11
P0_LEVELS = 5
P0G = 1 << P0_LEVELS
FG = 16


def _vsort(v):
    return jnp.sort(v)


def _vrev(v):
    return lax.rev(v, (0,))


def _reg_stages(vals, dists):
    n = len(vals)
    for d in dists:
        for s in range(0, n, 2 * d):
            for i in range(d):
                a = vals[s + i]
                b = vals[s + i + d]
                vals[s + i] = jnp.minimum(a, b)
                vals[s + i + d] = jnp.maximum(a, b)


def _reg_merge(vals):
    r = len(vals) // 2
    c = vals[:r] + [_vrev(v) for v in vals[r:][::-1]]
    dists = []
    d = r
    while d >= 1:
        dists.append(d)
        d //= 2
    _reg_stages(c, dists)
    return [_vsort(v) for v in c]


def _row_sort(buf):
    def vld(i):
        return buf[pl.ds(i * L, L)]

    def vst(i, v):
        buf[pl.ds(i * L, L)] = v

    def p0(m, c):
        base = m * P0G
        vals = [_vsort(vld(base + j)) for j in range(P0G)]
        for k in range(P0_LEVELS):
            sz = 1 << (k + 1)
            out = []
            for g in range(P0G // sz):
                out.extend(_reg_merge(vals[g * sz:(g + 1) * sz]))
            vals = out
        for j in range(P0G):
            vst(base + j, vals[j])
        return c

    lax.fori_loop(0, V // P0G, p0, 0)

    for k in range(P0_LEVELS, LOGV):
        R = 1 << k

        m = min(16, 1 << (k - 4))
        s = R // m
        ls = s.bit_length() - 1
        rs_dists = [1 << t for t in range((m.bit_length() - 1) - 1, -1, -1)]

        def refl(it, c, k=k, R=R, m=m, s=s, ls=ls, rs_dists=rs_dists):
            blk = it >> ls
            o = it & (s - 1)
            base = (blk << (k + 1)) + o
            top = (blk << (k + 1)) + 2 * R - 1 - o
            lo = [vld(base + j * s) for j in range(m)]
            hi = []
            for j in range(m):
                rb = _vrev(vld(top - j * s))
                a = lo[j]
                lo[j] = jnp.minimum(a, rb)
                hi.append(jnp.maximum(a, rb))
            hlist = [_vrev(hi[m - 1 - jp]) for jp in range(m)]
            _reg_stages(lo, rs_dists)
            _reg_stages(hlist, rs_dists)
            for j in range(m):
                vst(base + j * s, lo[j])
            for jp in range(m):
                vst(top - (m - 1 - jp) * s, hlist[jp])
            return c

        lax.fori_loop(0, V // (2 * m), refl, 0, unroll=2 if m <= 4 else 1)

        dists = []
        d = R // (2 * m)
        while d >= FG:
            dists.append(d)
            d //= 2

        gf, fdists = FG, [8, 4, 2, 1]
        if dists == [FG]:
            gf, fdists, dists = 2 * FG, [16, 8, 4, 2, 1], []

        while dists:
            take = 3 if len(dists) >= 3 else len(dists)
            chunk, dists = dists[:take], dists[take:]
            stride = chunk[-1]
            lss = stride.bit_length() - 1
            G = 1 << take
            block = 2 * chunk[0]
            lb = block.bit_length() - 1

            def fused(it, c, stride=stride, lss=lss, G=G, lb=lb, take=take):
                base = ((it >> lss) << lb) + (it & (stride - 1))
                g = [vld(base + j * stride) for j in range(G)]
                _reg_stages(g, [1 << t for t in range(take - 1, -1, -1)])
                for j in range(G):
                    vst(base + j * stride, g[j])
                return c

            lax.fori_loop(0, V // G, fused, 0, unroll=2 if G <= 4 else 1)

        def last(mm, c, gf=gf, fdists=fdists):
            base = mm * gf
            g = [vld(base + j) for j in range(gf)]
            _reg_stages(g, fdists)
            for j in range(gf):
                vst(base + j, _vsort(g[j]))
            return c

        lax.fori_loop(0, V // gf, last, 0, unroll=2 if gf == FG else 1)


def _sort_body(x_hbm, out_hbm, buf_a, buf_b, in_a, in_b, out_a, out_b):
    cid = lax.axis_index("c")
    sid = lax.axis_index("s")
    wid = sid * 2 + cid
    row0 = wid
    row1 = wid + NW

    cp_in0 = pltpu.make_async_copy(x_hbm.at[row0], buf_a, in_a)
    cp_in1 = pltpu.make_async_copy(x_hbm.at[row1], buf_b, in_b)
    cp_in0.start()
    cp_in1.start()

    cp_in0.wait()
    _row_sort(buf_a)
    cp_out0 = pltpu.make_async_copy(buf_a, out_hbm.at[row0], out_a)
    cp_out0.start()

    cp_in1.wait()
    _row_sort(buf_b)
    cp_out1 = pltpu.make_async_copy(buf_b, out_hbm.at[row1], out_b)
    cp_out1.start()

    cp_out0.wait()
    cp_out1.wait()


@jax.jit
def kernel(x):
    mesh = plsc.VectorSubcoreMesh(core_axis_name="c", subcore_axis_name="s")
    out = pl.kernel(
        _sort_body,
        out_type=jax.ShapeDtypeStruct((ROWS, N), jnp.float32),
        mesh=mesh,
        scratch_types=[
            pltpu.VMEM((N,), jnp.float32),
            pltpu.VMEM((N,), jnp.float32),
            pltpu.SemaphoreType.DMA,
            pltpu.SemaphoreType.DMA,
            pltpu.SemaphoreType.DMA,
            pltpu.SemaphoreType.DMA,
        ],
        compiler_params=pltpu.CompilerParams(needs_layout_passes=False),
    )(x)
    return out
